# Initial kernel scaffold; baseline (speedup 1.0000x reference)
import jax
import jax.numpy as jnp
from jax.experimental import pallas as pl


def kernel(embeddings, W, att_src, att_dst, bias):
    raise NotImplementedError("write your pallas kernel here")


# speedup vs baseline: 38.1834x; 38.1834x over previous
import jax
import jax.numpy as jnp
from jax.experimental import pallas as pl


def _gat_block_kernel(x_ref, w_ref, asrc_ref, adst_ref, bias_ref, out_ref):
    a, s_dim, _, two, d = x_ref.shape
    x = x_ref[...].reshape(a, s_dim, two, d)
    x = jnp.transpose(x, (0, 2, 1, 3)).reshape(a, s_dim * two, d)
    g = s_dim * two

    w = w_ref[...]
    h = jnp.dot(
        x.reshape(a * g, d), w, preferred_element_type=jnp.float32
    ).reshape(a, g, d)

    a_src = jnp.sum(h * asrc_ref[0][None, None, :], axis=-1)
    a_dst = jnp.sum(h * adst_ref[0][None, None, :], axis=-1)

    e = a_src[:, :, None] + jnp.transpose(a_dst)[None, :, :]
    e = jnp.where(e > 0, e, 0.2 * e)
    m = jnp.max(e, axis=0, keepdims=True)
    ex = jnp.exp(e - m)
    denom = jnp.sum(ex, axis=0, keepdims=True)
    alpha = ex / denom

    agg = jax.lax.dot_general(
        alpha, h, (((0,), (0,)), ((1,), (1,))),
        preferred_element_type=jnp.float32,
    )
    agg = jnp.transpose(agg, (1, 0, 2))

    out = agg + bias_ref[0][None, None, :] + x
    out_ref[...] = out.reshape(a, 1, g, d)


def kernel(embeddings, W, att_src, att_dst, bias):
    a, s, p, d = embeddings.shape
    emb5 = embeddings.reshape(a, s, p // 2, 2, d)
    grid = (p // 2,)
    out = pl.pallas_call(
        _gat_block_kernel,
        grid=grid,
        in_specs=[
            pl.BlockSpec((a, s, 1, 2, d), lambda u: (0, 0, u, 0, 0)),
            pl.BlockSpec((d, d), lambda u: (0, 0)),
            pl.BlockSpec((1, d), lambda u: (0, 0)),
            pl.BlockSpec((1, d), lambda u: (0, 0)),
            pl.BlockSpec((1, d), lambda u: (0, 0)),
        ],
        out_specs=pl.BlockSpec((a, 1, 2 * s, d), lambda u: (0, u, 0, 0)),
        out_shape=jax.ShapeDtypeStruct((a, p // 2, 2 * s, d), jnp.float32),
    )(
        emb5,
        W,
        att_src.reshape(1, d),
        att_dst.reshape(1, d),
        bias.reshape(1, d),
    )
    return out


# speedup vs baseline: 50.5705x; 1.3244x over previous
import jax
import jax.numpy as jnp
from jax.experimental import pallas as pl


def _gat_block_kernel(x_ref, w_ref, asrc_ref, adst_ref, bias_ref, out_ref):
    a, s_dim, _, two, d = x_ref.shape
    g = s_dim * two
    xb = x_ref[...].reshape(a, s_dim, two, d)
    xm = jnp.transpose(xb, (2, 1, 0, 3)).reshape(g, a, d)

    w = w_ref[...]
    h = jnp.dot(
        xm.reshape(g * a, d), w, preferred_element_type=jnp.float32
    ).reshape(g, a, d)

    a_src = jnp.sum(h * asrc_ref[0][None, None, :], axis=-1)
    a_dst = jnp.sum(h * adst_ref[0][None, None, :], axis=-1)

    e = a_src[:, :, None] + a_dst[:, None, :]
    e = jnp.where(e > 0, e, 0.2 * e)
    m = jnp.max(e, axis=1, keepdims=True)
    ex = jnp.exp(e - m)
    denom = jnp.sum(ex, axis=1, keepdims=True)
    alpha = ex / denom

    agg = jax.lax.dot_general(
        alpha, h, (((1,), (1,)), ((0,), (0,))),
        preferred_element_type=jnp.float32,
    )

    out = agg + bias_ref[0][None, None, :] + xm
    out = jnp.transpose(out, (1, 0, 2))
    out_ref[...] = out.reshape(a, 1, g, d)


def kernel(embeddings, W, att_src, att_dst, bias):
    a, s, p, d = embeddings.shape
    emb5 = embeddings.reshape(a, s, p // 2, 2, d)
    grid = (p // 2,)
    out = pl.pallas_call(
        _gat_block_kernel,
        grid=grid,
        in_specs=[
            pl.BlockSpec((a, s, 1, 2, d), lambda u: (0, 0, u, 0, 0)),
            pl.BlockSpec((d, d), lambda u: (0, 0)),
            pl.BlockSpec((1, d), lambda u: (0, 0)),
            pl.BlockSpec((1, d), lambda u: (0, 0)),
            pl.BlockSpec((1, d), lambda u: (0, 0)),
        ],
        out_specs=pl.BlockSpec((a, 1, 2 * s, d), lambda u: (0, u, 0, 0)),
        out_shape=jax.ShapeDtypeStruct((a, p // 2, 2 * s, d), jnp.float32),
    )(
        emb5,
        W,
        att_src.reshape(1, d),
        att_dst.reshape(1, d),
        bias.reshape(1, d),
    )
    return out


# speedup vs baseline: 62.0787x; 1.2276x over previous
import jax
import jax.numpy as jnp
from jax.experimental import pallas as pl


def _gat_block_kernel(x_ref, w_ref, asrc_ref, adst_ref, bias_ref, out_ref):
    a, s_dim, _, two, d = x_ref.shape
    g = s_dim * two
    xb = x_ref[...].reshape(a, s_dim, two, d)
    xm = jnp.transpose(xb, (2, 1, 0, 3)).reshape(g, a, d)

    w = w_ref[...]
    h = jnp.dot(
        xm.reshape(g * a, d), w, preferred_element_type=jnp.float32
    ).reshape(g, a, d)

    a_src = jnp.sum(h * asrc_ref[0][None, None, :], axis=-1)
    a_dst = jnp.sum(h * adst_ref[0][None, None, :], axis=-1)

    e = a_src[:, :, None] + a_dst[:, None, :]
    e = jnp.where(e > 0, e, 0.2 * e)
    m = jnp.max(e, axis=1, keepdims=True)
    ex = jnp.exp(e - m)
    denom = jnp.sum(ex, axis=1, keepdims=True)
    alpha = ex / denom

    agg = jax.lax.dot_general(
        alpha, h, (((1,), (1,)), ((0,), (0,))),
        preferred_element_type=jnp.float32,
    )

    out = agg + bias_ref[0][None, None, :] + xm
    out = jnp.transpose(out, (1, 0, 2))
    out_ref[...] = out.reshape(a, 2, g // 2, d)


def kernel(embeddings, W, att_src, att_dst, bias):
    a, s, p, d = embeddings.shape
    emb5 = embeddings.reshape(a, s, p // 4, 4, d)
    grid = (p // 4,)
    out = pl.pallas_call(
        _gat_block_kernel,
        grid=grid,
        in_specs=[
            pl.BlockSpec((a, s, 1, 4, d), lambda u: (0, 0, u, 0, 0)),
            pl.BlockSpec((d, d), lambda u: (0, 0)),
            pl.BlockSpec((1, d), lambda u: (0, 0)),
            pl.BlockSpec((1, d), lambda u: (0, 0)),
            pl.BlockSpec((1, d), lambda u: (0, 0)),
        ],
        out_specs=pl.BlockSpec((a, 2, 2 * s, d), lambda u: (0, u, 0, 0)),
        out_shape=jax.ShapeDtypeStruct((a, p // 2, 2 * s, d), jnp.float32),
    )(
        emb5,
        W,
        att_src.reshape(1, d),
        att_dst.reshape(1, d),
        bias.reshape(1, d),
    )
    return out


# speedup vs baseline: 68.3171x; 1.1005x over previous
import jax
import jax.numpy as jnp
from jax.experimental import pallas as pl


def _gat_block_kernel(x_ref, w_ref, asrc_ref, adst_ref, bias_ref, out_ref):
    a, s_dim, _, two, d = x_ref.shape
    g = s_dim * two
    xb = x_ref[...].reshape(a, s_dim, two, d)
    xm = jnp.transpose(xb, (2, 1, 0, 3)).reshape(g, a, d)

    w = w_ref[...]
    h = jnp.dot(
        xm.reshape(g * a, d), w, preferred_element_type=jnp.float32
    ).reshape(g, a, d)

    a_src = jnp.sum(h * asrc_ref[0][None, None, :], axis=-1)
    a_dst = jnp.sum(h * adst_ref[0][None, None, :], axis=-1)

    e = a_src[:, :, None] + a_dst[:, None, :]
    e = jnp.where(e > 0, e, 0.2 * e)
    m = jnp.max(e, axis=1, keepdims=True)
    ex = jnp.exp(e - m)
    denom = jnp.sum(ex, axis=1, keepdims=True)
    alpha = ex / denom

    agg = jax.lax.dot_general(
        alpha, h, (((1,), (1,)), ((0,), (0,))),
        preferred_element_type=jnp.float32,
    )

    out = agg + bias_ref[0][None, None, :] + xm
    out = jnp.transpose(out, (1, 0, 2))
    out_ref[...] = out.reshape(a, 4, g // 4, d)


def kernel(embeddings, W, att_src, att_dst, bias):
    a, s, p, d = embeddings.shape
    emb5 = embeddings.reshape(a, s, p // 8, 8, d)
    grid = (p // 8,)
    out = pl.pallas_call(
        _gat_block_kernel,
        grid=grid,
        in_specs=[
            pl.BlockSpec((a, s, 1, 8, d), lambda u: (0, 0, u, 0, 0)),
            pl.BlockSpec((d, d), lambda u: (0, 0)),
            pl.BlockSpec((1, d), lambda u: (0, 0)),
            pl.BlockSpec((1, d), lambda u: (0, 0)),
            pl.BlockSpec((1, d), lambda u: (0, 0)),
        ],
        out_specs=pl.BlockSpec((a, 4, 2 * s, d), lambda u: (0, u, 0, 0)),
        out_shape=jax.ShapeDtypeStruct((a, p // 2, 2 * s, d), jnp.float32),
    )(
        emb5,
        W,
        att_src.reshape(1, d),
        att_dst.reshape(1, d),
        bias.reshape(1, d),
    )
    return out
